# baseline (speedup 1.0000x reference)
import functools

import jax
import jax.numpy as jnp
from jax import lax
from jax.experimental import pallas as pl
from jax.experimental.pallas import tpu as pltpu
from jax.experimental.pallas import tpu_sc as plsc

_B, _N, _D = 16, 64, 16
_M = _N * _D
_HALF_BLKS = _N // 2
_ROWS = 32
_CHUNKS = (_HALF_BLKS * _D) // _ROWS


def _sc_body(x_hbm, out_hbm, staged, buf0, buf1, sem0, sem1):
    nc = 2
    wid = lax.axis_index("s") * nc + lax.axis_index("c")
    b = wid // 2
    h = wid % 2
    blk0 = h * _HALF_BLKS
    row0 = h * (_HALF_BLKS * _D)

    pltpu.sync_copy(x_hbm.at[b, pl.ds(blk0, _HALF_BLKS)], staged)

    zero = jnp.zeros((_D,), jnp.float32)

    def _zero_row(r, carry):
        for k in range(_M // _D):
            buf0[r, pl.ds(k * _D, _D)] = zero
            buf1[r, pl.ds(k * _D, _D)] = zero
        return carry

    lax.fori_loop(0, _ROWS, _zero_row, 0)

    bufs = (buf0, buf1)
    sems = (sem0, sem1)
    copies = [None, None]
    for c in range(_CHUNKS):
        p = c % 2
        buf = bufs[p]
        if c >= 2:
            copies[p].wait()
            for jj in range(2):
                col_old = h * (_HALF_BLKS * _D) + ((c - 2) * 2 + jj) * _D
                for r in range(_D):
                    buf[jj * _D + r, pl.ds(col_old, _D)] = zero
        for jj in range(2):
            col = h * (_HALF_BLKS * _D) + (c * 2 + jj) * _D
            for r in range(_D):
                buf[jj * _D + r, pl.ds(col, _D)] = staged[c * 2 + jj, r, :]
        copies[p] = pltpu.async_copy(
            buf, out_hbm.at[b, pl.ds(row0 + c * _ROWS, _ROWS)], sems[p])
    copies[0].wait()
    copies[1].wait()


def kernel(input):
    mesh = plsc.VectorSubcoreMesh(core_axis_name="c", subcore_axis_name="s")
    run = functools.partial(
        pl.kernel,
        mesh=mesh,
        out_type=jax.ShapeDtypeStruct((_B, _M, _M), jnp.float32),
        scratch_types=[
            pltpu.VMEM((_HALF_BLKS, _D, _D), jnp.float32),
            pltpu.VMEM((_ROWS, _M), jnp.float32),
            pltpu.VMEM((_ROWS, _M), jnp.float32),
            pltpu.SemaphoreType.DMA,
            pltpu.SemaphoreType.DMA,
        ],
    )(_sc_body)
    return run(input)
